# Initial kernel scaffold; baseline (speedup 1.0000x reference)
import jax
import jax.numpy as jnp
from jax.experimental import pallas as pl


def kernel(pix_to_face, zbuf, bary_coords, faces, verts, verts_rgb):
    raise NotImplementedError("write your pallas kernel here")


# speedup vs baseline: 235.2641x; 235.2641x over previous
import functools

import jax
import jax.numpy as jnp
from jax import lax
from jax.experimental import pallas as pl
from jax.experimental.pallas import tpu as pltpu
from jax.experimental.pallas import tpu_sc as plsc

H = W = 512
P = H * W
NC = 2
NS = 16
NW = NC * NS
PER_W = P // NW
CH = 2048
N_CH = PER_W // CH
LANES = 16


def _shader_body(p2f_hbm, b0_hbm, b1_hbm, b2_hbm,
                 f0_hbm, f1_hbm, f2_hbm,
                 tr_hbm, tg_hbm, tb_hbm,
                 outr_hbm, outg_hbm, outb_hbm,
                 f_v, cidx, v0, v1, v2,
                 b0v, b1v, b2v,
                 r0, r1, r2, g0, g1, g2, bl0, bl1, bl2,
                 outr, outg, outb, sem):
    wid = lax.axis_index("c") * NS + lax.axis_index("s")

    for ci in range(N_CH):
        base = wid * PER_W + ci * CH

        pltpu.sync_copy(p2f_hbm.at[pl.ds(base, CH)], f_v)
        pltpu.sync_copy(b0_hbm.at[pl.ds(base, CH)], b0v)
        pltpu.sync_copy(b1_hbm.at[pl.ds(base, CH)], b1v)
        pltpu.sync_copy(b2_hbm.at[pl.ds(base, CH)], b2v)

        def clip_body(i, _):
            sl = pl.ds(i * LANES, LANES)
            cidx[sl] = jnp.maximum(f_v[sl], 0)
            return 0

        lax.fori_loop(0, CH // LANES, clip_body, 0)

        c0 = pltpu.async_copy(f0_hbm.at[cidx], v0, sem)
        c1 = pltpu.async_copy(f1_hbm.at[cidx], v1, sem)
        c2 = pltpu.async_copy(f2_hbm.at[cidx], v2, sem)
        c0.wait()
        c1.wait()
        c2.wait()

        copies = []
        for vtx, dsts in ((v0, (r0, g0, bl0)), (v1, (r1, g1, bl1)),
                          (v2, (r2, g2, bl2))):
            for tbl, dst in zip((tr_hbm, tg_hbm, tb_hbm), dsts):
                copies.append(pltpu.async_copy(tbl.at[vtx], dst, sem))
        for c in copies:
            c.wait()

        def cmp_body(i, _):
            sl = pl.ds(i * LANES, LANES)
            valid = f_v[sl] >= 0
            b0x = b0v[sl]
            b1x = b1v[sl]
            b2x = b2v[sl]
            for chans, out_ref in (((r0, r1, r2), outr),
                                   ((g0, g1, g2), outg),
                                   ((bl0, bl1, bl2), outb)):
                acc = b0x * chans[0][sl] + b1x * chans[1][sl] + b2x * chans[2][sl]
                out_ref[sl] = jnp.where(valid, acc, 1.0)
            return 0

        lax.fori_loop(0, CH // LANES, cmp_body, 0)

        pltpu.sync_copy(outr, outr_hbm.at[pl.ds(base, CH)])
        pltpu.sync_copy(outg, outg_hbm.at[pl.ds(base, CH)])
        pltpu.sync_copy(outb, outb_hbm.at[pl.ds(base, CH)])


@jax.jit
def _shade(p2f, b0, b1, b2, f0, f1, f2, tr, tg, tb):
    mesh = plsc.VectorSubcoreMesh(core_axis_name="c", subcore_axis_name="s")
    plane = jax.ShapeDtypeStruct((P,), jnp.float32)
    ivec = pltpu.VMEM((CH,), jnp.int32)
    fvec = pltpu.VMEM((CH,), jnp.float32)
    run = functools.partial(
        pl.kernel,
        mesh=mesh,
        out_type=(plane, plane, plane),
        scratch_types=[ivec] * 5 + [fvec] * 15 + [pltpu.SemaphoreType.DMA],
    )(_shader_body)
    return run(p2f, b0, b1, b2, f0, f1, f2, tr, tg, tb)


def kernel(pix_to_face, zbuf, bary_coords, faces, verts, verts_rgb):
    del zbuf, verts
    n = pix_to_face.shape[0]
    p2f = pix_to_face[..., 0].reshape(P)
    bary = bary_coords[..., 0, :].reshape(P, 3)
    ft = faces.T
    vt = verts_rgb.T
    r, g, b = _shade(p2f, bary[:, 0], bary[:, 1], bary[:, 2],
                     ft[0], ft[1], ft[2], vt[0], vt[1], vt[2])
    rgb = jnp.stack([r, g, b], axis=-1)
    alpha = jnp.ones((P, 1), jnp.float32)
    return jnp.concatenate([rgb, alpha], axis=-1).reshape(n, H, W, 4)


# speedup vs baseline: 427.0135x; 1.8150x over previous
import functools

import jax
import jax.numpy as jnp
from jax import lax
from jax.experimental import pallas as pl
from jax.experimental.pallas import tpu as pltpu
from jax.experimental.pallas import tpu_sc as plsc

H = W = 512
P = H * W
NC = 2
NS = 16
NW = NC * NS
PER_W = P // NW
CH = 2048
N_CH = PER_W // CH
LANES = 16


def _shader_body(p2f_hbm, b0_hbm, b1_hbm, b2_hbm,
                 f0_hbm, f1_hbm, f2_hbm,
                 tr_hbm, tg_hbm, tb_hbm,
                 outr_hbm, outg_hbm, outb_hbm,
                 f0_s, f1_s, f2_s, tr_s, tg_s, tb_s,
                 f_v, cidx, v0, v1, v2,
                 b0v, b1v, b2v,
                 r0, r1, r2, g0, g1, g2, bl0, bl1, bl2,
                 outr, outg, outb, sem):
    sid = lax.axis_index("s")
    wid = lax.axis_index("c") * NS + sid

    @pl.when(sid == 0)
    def _stage():
        for src, dst in ((f0_hbm, f0_s), (f1_hbm, f1_s), (f2_hbm, f2_s),
                         (tr_hbm, tr_s), (tg_hbm, tg_s), (tb_hbm, tb_s)):
            pltpu.sync_copy(src, dst)

    plsc.subcore_barrier()

    for ci in range(N_CH):
        base = wid * PER_W + ci * CH

        pltpu.sync_copy(p2f_hbm.at[pl.ds(base, CH)], f_v)
        pltpu.sync_copy(b0_hbm.at[pl.ds(base, CH)], b0v)
        pltpu.sync_copy(b1_hbm.at[pl.ds(base, CH)], b1v)
        pltpu.sync_copy(b2_hbm.at[pl.ds(base, CH)], b2v)

        def clip_body(i, _):
            sl = pl.ds(i * LANES, LANES)
            cidx[sl] = jnp.maximum(f_v[sl], 0)
            return 0

        lax.fori_loop(0, CH // LANES, clip_body, 0)

        c0 = pltpu.async_copy(f0_s.at[cidx], v0, sem)
        c1 = pltpu.async_copy(f1_s.at[cidx], v1, sem)
        c2 = pltpu.async_copy(f2_s.at[cidx], v2, sem)
        c0.wait()
        c1.wait()
        c2.wait()

        copies = []
        for vtx, dsts in ((v0, (r0, g0, bl0)), (v1, (r1, g1, bl1)),
                          (v2, (r2, g2, bl2))):
            for tbl, dst in zip((tr_s, tg_s, tb_s), dsts):
                copies.append(pltpu.async_copy(tbl.at[vtx], dst, sem))
        for c in copies:
            c.wait()

        def cmp_body(i, _):
            sl = pl.ds(i * LANES, LANES)
            valid = f_v[sl] >= 0
            b0x = b0v[sl]
            b1x = b1v[sl]
            b2x = b2v[sl]
            for chans, out_ref in (((r0, r1, r2), outr),
                                   ((g0, g1, g2), outg),
                                   ((bl0, bl1, bl2), outb)):
                acc = b0x * chans[0][sl] + b1x * chans[1][sl] + b2x * chans[2][sl]
                out_ref[sl] = jnp.where(valid, acc, 1.0)
            return 0

        lax.fori_loop(0, CH // LANES, cmp_body, 0)

        pltpu.sync_copy(outr, outr_hbm.at[pl.ds(base, CH)])
        pltpu.sync_copy(outg, outg_hbm.at[pl.ds(base, CH)])
        pltpu.sync_copy(outb, outb_hbm.at[pl.ds(base, CH)])


@jax.jit
def _shade(p2f, b0, b1, b2, f0, f1, f2, tr, tg, tb):
    mesh = plsc.VectorSubcoreMesh(core_axis_name="c", subcore_axis_name="s")
    plane = jax.ShapeDtypeStruct((P,), jnp.float32)
    ivec = pltpu.VMEM((CH,), jnp.int32)
    fvec = pltpu.VMEM((CH,), jnp.float32)
    F = f0.shape[0]
    V = tr.shape[0]
    shared = ([pltpu.VMEM_SHARED((F,), jnp.int32)] * 3
              + [pltpu.VMEM_SHARED((V,), jnp.float32)] * 3)
    run = functools.partial(
        pl.kernel,
        mesh=mesh,
        out_type=(plane, plane, plane),
        scratch_types=shared + [ivec] * 5 + [fvec] * 15
        + [pltpu.SemaphoreType.DMA],
    )(_shader_body)
    return run(p2f, b0, b1, b2, f0, f1, f2, tr, tg, tb)


def kernel(pix_to_face, zbuf, bary_coords, faces, verts, verts_rgb):
    del zbuf, verts
    n = pix_to_face.shape[0]
    p2f = pix_to_face[..., 0].reshape(P)
    bary = bary_coords[..., 0, :].reshape(P, 3)
    ft = faces.T
    vt = verts_rgb.T
    r, g, b = _shade(p2f, bary[:, 0], bary[:, 1], bary[:, 2],
                     ft[0], ft[1], ft[2], vt[0], vt[1], vt[2])
    rgb = jnp.stack([r, g, b], axis=-1)
    alpha = jnp.ones((P, 1), jnp.float32)
    return jnp.concatenate([rgb, alpha], axis=-1).reshape(n, H, W, 4)


# speedup vs baseline: 481.1704x; 1.1268x over previous
import functools

import jax
import jax.numpy as jnp
from jax import lax
from jax.experimental import pallas as pl
from jax.experimental.pallas import tpu as pltpu
from jax.experimental.pallas import tpu_sc as plsc

H = W = 512
P = H * W
NC = 2
NS = 16
NW = NC * NS
PER_W = P // NW
CH = 1024
N_CH = PER_W // CH
LANES = 16

_N_IVEC = 5
_N_FVEC = 15


def _shader_body(p2f_hbm, b0_hbm, b1_hbm, b2_hbm,
                 f0_hbm, f1_hbm, f2_hbm,
                 tr_hbm, tg_hbm, tb_hbm,
                 outr_hbm, outg_hbm, outb_hbm,
                 *refs):
    f0_s, f1_s, f2_s, tr_s, tg_s, tb_s = refs[:6]
    per_set = _N_IVEC + _N_FVEC + 1
    sets = [refs[6 + i * per_set:6 + (i + 1) * per_set] for i in range(2)]

    sid = lax.axis_index("s")
    wid = lax.axis_index("c") * NS + sid

    def base_of(ci):
        return wid * PER_W + ci * CH

    def load_clip(ci, S):
        f_v, cidx = S[0], S[1]
        b0v, b1v, b2v = S[5], S[6], S[7]
        base = base_of(ci)
        pltpu.sync_copy(p2f_hbm.at[pl.ds(base, CH)], f_v)
        pltpu.sync_copy(b0_hbm.at[pl.ds(base, CH)], b0v)
        pltpu.sync_copy(b1_hbm.at[pl.ds(base, CH)], b1v)
        pltpu.sync_copy(b2_hbm.at[pl.ds(base, CH)], b2v)

        def clip_body(i, _):
            sl = pl.ds(i * LANES, LANES)
            cidx[sl] = jnp.maximum(f_v[sl], 0)
            return 0

        lax.fori_loop(0, CH // LANES, clip_body, 0)

    def fire_faces(S):
        cidx, sem = S[1], S[-1]
        return [pltpu.async_copy(tbl.at[cidx], S[2 + j], sem)
                for j, tbl in enumerate((f0_s, f1_s, f2_s))]

    def fire_rgb(S):
        sem = S[-1]
        copies = []
        for j in range(3):
            vtx = S[2 + j]
            for k, tbl in enumerate((tr_s, tg_s, tb_s)):
                copies.append(pltpu.async_copy(tbl.at[vtx], S[8 + 3 * j + k],
                                               sem))
        return copies

    def compute_store(ci, S):
        f_v = S[0]
        b0v, b1v, b2v = S[5], S[6], S[7]
        r0, g0, bl0 = S[8], S[9], S[10]
        r1, g1, bl1 = S[11], S[12], S[13]
        r2, g2, bl2 = S[14], S[15], S[16]
        outr, outg, outb = S[17], S[18], S[19]

        def cmp_body(i, _):
            sl = pl.ds(i * LANES, LANES)
            valid = f_v[sl] >= 0
            b0x = b0v[sl]
            b1x = b1v[sl]
            b2x = b2v[sl]
            for chans, out_ref in (((r0, r1, r2), outr),
                                   ((g0, g1, g2), outg),
                                   ((bl0, bl1, bl2), outb)):
                acc = (b0x * chans[0][sl] + b1x * chans[1][sl]
                       + b2x * chans[2][sl])
                out_ref[sl] = jnp.where(valid, acc, 1.0)
            return 0

        lax.fori_loop(0, CH // LANES, cmp_body, 0)

        base = base_of(ci)
        pltpu.sync_copy(outr, outr_hbm.at[pl.ds(base, CH)])
        pltpu.sync_copy(outg, outg_hbm.at[pl.ds(base, CH)])
        pltpu.sync_copy(outb, outb_hbm.at[pl.ds(base, CH)])

    @pl.when(sid == 0)
    def _stage():
        for src, dst in ((f0_hbm, f0_s), (f1_hbm, f1_s), (f2_hbm, f2_s),
                         (tr_hbm, tr_s), (tg_hbm, tg_s), (tb_hbm, tb_s)):
            pltpu.sync_copy(src, dst)

    load_clip(0, sets[0])
    plsc.subcore_barrier()

    faces_inflight = fire_faces(sets[0])
    for ci in range(N_CH):
        S = sets[ci % 2]
        T = sets[(ci + 1) % 2]
        for c in faces_inflight:
            c.wait()
        rgb_inflight = fire_rgb(S)
        if ci + 1 < N_CH:
            load_clip(ci + 1, T)
            faces_inflight = fire_faces(T)
        for c in rgb_inflight:
            c.wait()
        compute_store(ci, S)


@jax.jit
def _shade(p2f, b0, b1, b2, f0, f1, f2, tr, tg, tb):
    mesh = plsc.VectorSubcoreMesh(core_axis_name="c", subcore_axis_name="s")
    plane = jax.ShapeDtypeStruct((P,), jnp.float32)
    ivec = pltpu.VMEM((CH,), jnp.int32)
    fvec = pltpu.VMEM((CH,), jnp.float32)
    F = f0.shape[0]
    V = tr.shape[0]
    shared = ([pltpu.VMEM_SHARED((F,), jnp.int32)] * 3
              + [pltpu.VMEM_SHARED((V,), jnp.float32)] * 3)
    buf_set = [ivec] * _N_IVEC + [fvec] * _N_FVEC + [pltpu.SemaphoreType.DMA]
    run = functools.partial(
        pl.kernel,
        mesh=mesh,
        out_type=(plane, plane, plane),
        scratch_types=shared + buf_set * 2,
    )(_shader_body)
    return run(p2f, b0, b1, b2, f0, f1, f2, tr, tg, tb)


def kernel(pix_to_face, zbuf, bary_coords, faces, verts, verts_rgb):
    del zbuf, verts
    n = pix_to_face.shape[0]
    p2f = pix_to_face[..., 0].reshape(P)
    bary = bary_coords[..., 0, :].reshape(P, 3)
    ft = faces.T
    vt = verts_rgb.T
    r, g, b = _shade(p2f, bary[:, 0], bary[:, 1], bary[:, 2],
                     ft[0], ft[1], ft[2], vt[0], vt[1], vt[2])
    rgb = jnp.stack([r, g, b], axis=-1)
    alpha = jnp.ones((P, 1), jnp.float32)
    return jnp.concatenate([rgb, alpha], axis=-1).reshape(n, H, W, 4)
